# baseline (speedup 1.0000x reference)
import functools

import jax
import jax.numpy as jnp
from jax import lax
from jax.experimental import pallas as pl
from jax.experimental.pallas import tpu as pltpu
from jax.experimental.pallas import tpu_sc as plsc

NC, NS = 2, 16
NW = NC * NS
B = 16384
D = 128
CHUNK = 128
ROWS_PER_W = B // NW
N_CHUNKS = ROWS_PER_W // CHUNK


NBUF = 6


@functools.lru_cache(maxsize=None)
def _make_sc_gather(n):
    rows_per_w = n // NW
    n_chunks = rows_per_w // CHUNK

    def body(user_table, product_table, uid2d, pid2d, nid2d,
             u_out, p_out, n_out,
             idx_u, idx_p, idx_n, b0, b1, b2, b3, b4, b5,
             g0, g1, g2, g3, g4, g5, t0, t1, t2, t3, t4, t5):
        bufs = (b0, b1, b2, b3, b4, b5)
        gsems = (g0, g1, g2, g3, g4, g5)
        ssems = (t0, t1, t2, t3, t4, t5)
        wid = lax.axis_index("s") * NC + lax.axis_index("c")
        base = wid * rows_per_w
        crow = wid * n_chunks
        pltpu.sync_copy(uid2d.at[pl.ds(crow, n_chunks)], idx_u)
        pltpu.sync_copy(pid2d.at[pl.ds(crow, n_chunks)], idx_p)
        pltpu.sync_copy(nid2d.at[pl.ds(crow, n_chunks)], idx_n)
        tasks = []
        for tbl, idx, out in ((user_table, idx_u, u_out),
                              (product_table, idx_p, p_out),
                              (product_table, idx_n, n_out)):
            for j in range(n_chunks):
                tasks.append((tbl, idx, j, out))
        T = len(tasks)
        LB = NBUF - 1
        gathers = [None] * T
        stores = [None] * T
        for t in range(T + LB):
            if t < T:
                tbl, idx, j, out = tasks[t]
                nb = t % NBUF
                if t >= NBUF:
                    stores[t - NBUF].wait()
                gathers[t] = pltpu.async_copy(tbl.at[idx.at[j]], bufs[nb],
                                              gsems[nb])
            u = t - LB
            if 0 <= u < T:
                _, _, j, out = tasks[u]
                gathers[u].wait()
                stores[u] = pltpu.async_copy(
                    bufs[u % NBUF], out.at[pl.ds(base + j * CHUNK, CHUNK)],
                    ssems[u % NBUF])
        for t in range(max(0, T - NBUF), T):
            stores[t].wait()

    return functools.partial(
        pl.kernel,
        out_type=(
            jax.ShapeDtypeStruct((n, D), jnp.float32),
            jax.ShapeDtypeStruct((n, D), jnp.float32),
            jax.ShapeDtypeStruct((n, D), jnp.float32),
        ),
        mesh=plsc.VectorSubcoreMesh(core_axis_name="c", subcore_axis_name="s",
                                    num_cores=NC, num_subcores=NS),
        scratch_types=(
            [pltpu.VMEM((n_chunks, CHUNK), jnp.int32)] * 3
            + [pltpu.VMEM((CHUNK, D), jnp.float32)] * NBUF
            + [pltpu.SemaphoreType.DMA] * (2 * NBUF)
        ),
    )(body)


BLK = 2048


def _tc_score_body(pos_ref, com_ref, usr_ref, neg_ref, w1_ref, w2_ref, s_ref,
                   sp_ref, sn_ref):
    dn = (((1,), (1,)), ((), ()))
    dncol = (((1,), (0,)), ((), ()))
    fused = (lax.dot_general(pos_ref[...], w1_ref[...], dn,
                             preferred_element_type=jnp.float32)
             + lax.dot_general(com_ref[...], w2_ref[...], dn,
                               preferred_element_type=jnp.float32))
    usr = usr_ref[...]
    out2 = (
        lax.dot_general(usr * fused, s_ref[0:D, :], dncol,
                        preferred_element_type=jnp.float32)
        + lax.dot_general(usr * neg_ref[...], s_ref[D:2 * D, :], dncol,
                          preferred_element_type=jnp.float32)
        + lax.dot_general(usr, s_ref[2 * D:3 * D, :], dncol,
                          preferred_element_type=jnp.float32))
    sp_ref[...] = out2[:, 0]
    sn_ref[...] = out2[:, 1]


def _tc_score(pos_emb, comment, user_emb, neg_emb, w1, w2, smat):
    n = pos_emb.shape[0]
    grid = (n // BLK,)
    row_spec = pl.BlockSpec((BLK, D), lambda i: (i, 0))
    full_spec = pl.BlockSpec((D, D), lambda i: (0, 0))
    return pl.pallas_call(
        _tc_score_body,
        grid=grid,
        in_specs=[row_spec, row_spec, row_spec, row_spec,
                  full_spec, full_spec,
                  pl.BlockSpec((3 * D, 2), lambda i: (0, 0))],
        out_specs=[pl.BlockSpec((BLK,), lambda i: (i,)),
                   pl.BlockSpec((BLK,), lambda i: (i,))],
        out_shape=[jax.ShapeDtypeStruct((n,), jnp.float32),
                   jax.ShapeDtypeStruct((n,), jnp.float32)],
    )(pos_emb, comment, user_emb, neg_emb, w1, w2, smat)


def kernel(user_ids, positive_product_ids, negative_product_ids,
           positive_comment_embeddings, user_table, product_table, W, b):
    w1 = W[:, :D]
    w2 = W[:, D:]
    smat = jnp.concatenate([
        jnp.tile(jnp.array([[1.0, 0.0]], jnp.float32), (D, 1)),
        jnp.tile(jnp.array([[0.0, 1.0]], jnp.float32), (D, 1)),
        jnp.stack([b, jnp.zeros_like(b)], axis=1),
    ], axis=0)
    H = B // 2
    sc = _make_sc_gather(H)
    sps, sns = [], []
    embs = []
    for h in range(2):
        sl = slice(h * H, (h + 1) * H)
        embs.append(sc(
            user_table, product_table,
            user_ids[sl].reshape(H // CHUNK, CHUNK),
            positive_product_ids[sl].reshape(H // CHUNK, CHUNK),
            negative_product_ids[sl].reshape(H // CHUNK, CHUNK)))
    for h in range(2):
        sl = slice(h * H, (h + 1) * H)
        user_emb, pos_emb, neg_emb = embs[h]
        sp, sn = _tc_score(pos_emb, positive_comment_embeddings[sl],
                           user_emb, neg_emb, w1, w2, smat)
        sps.append(sp)
        sns.append(sn)
    return (jnp.concatenate(sps), jnp.concatenate(sns))


# speedup vs baseline: 1.0517x; 1.0517x over previous
import functools

import jax
import jax.numpy as jnp
from jax import lax
from jax.experimental import pallas as pl
from jax.experimental.pallas import tpu as pltpu
from jax.experimental.pallas import tpu_sc as plsc

NC, NS = 2, 16
NW = NC * NS
B = 16384
D = 128
CHUNK = 128
ROWS_PER_W = B // NW
N_CHUNKS = ROWS_PER_W // CHUNK
NBUF = 6
NSEG = D // 16


def _lane_perm(v, sh):
    perm = lax.bitwise_xor(lax.iota(jnp.int32, 16), sh)
    return lax.gather(
        v, perm[:, None],
        dimension_numbers=lax.GatherDimensionNumbers(
            offset_dims=(), collapsed_slice_dims=(0,), start_index_map=(0,)),
        slice_sizes=(1,),
        mode=lax.GatherScatterMode.PROMISE_IN_BOUNDS)


def _sc_body(user_table, product_table, uid2d, pid2d, nid2d,
             u_out, p_out, sneg_out,
             idx_u, idx_p, idx_n, scores_v,
             b0, b1, b2, b3, b4, b5,
             g0, g1, g2, g3, g4, g5, t0, t1, t2, t3, t4, t5):
    bufs = (b0, b1, b2, b3, b4, b5)
    gsems = (g0, g1, g2, g3, g4, g5)
    ssems = (t0, t1, t2, t3, t4, t5)
    wid = lax.axis_index("s") * NC + lax.axis_index("c")
    base = wid * ROWS_PER_W
    crow = wid * N_CHUNKS
    du = pltpu.async_copy(uid2d.at[pl.ds(crow, N_CHUNKS)], idx_u, t0)
    dn = pltpu.async_copy(nid2d.at[pl.ds(crow, N_CHUNKS)], idx_n, t1)
    dp = pltpu.async_copy(pid2d.at[pl.ds(crow, N_CHUNKS)], idx_p, t2)
    du.wait()
    dn.wait()
    dp.wait()

    def slots(j):
        return (3 * j) % NBUF, (3 * j + 1) % NBUF, (3 * j + 2) % NBUF

    def issue(j):
        su, sn, sp = slots(j)
        gu = pltpu.async_copy(user_table.at[idx_u.at[j]], bufs[su], gsems[su])
        gn = pltpu.async_copy(product_table.at[idx_n.at[j]], bufs[sn],
                              gsems[sn])
        gp = pltpu.async_copy(product_table.at[idx_p.at[j]], bufs[sp],
                              gsems[sp])
        return gu, gn, gp

    gathers = [None] * N_CHUNKS
    stores = [None] * N_CHUNKS
    gathers[0] = issue(0)
    for j in range(N_CHUNKS):
        if j + 1 < N_CHUNKS:
            if j >= 1:
                stores[j - 1][0].wait()
                stores[j - 1][1].wait()
            gathers[j + 1] = issue(j + 1)
        su, sn, sp = slots(j)
        gu, gn, gp = gathers[j]
        gu.wait()
        gp.wait()
        st_u = pltpu.async_copy(
            bufs[su], u_out.at[pl.ds(base + j * CHUNK, CHUNK)], ssems[su])
        st_p = pltpu.async_copy(
            bufs[sp], p_out.at[pl.ds(base + j * CHUNK, CHUNK)], ssems[sp])
        stores[j] = (st_u, st_p)
        gn.wait()
        bu = bufs[su]
        bn = bufs[sn]
        lanes = lax.iota(jnp.int32, 16)
        joff = j * CHUNK

        def row_total(r):
            acc = bu[r, pl.ds(0, 16)] * bn[r, pl.ds(0, 16)]
            for s in range(1, NSEG):
                acc += bu[r, pl.ds(16 * s, 16)] * bn[r, pl.ds(16 * s, 16)]
            for sh in (8, 4, 2, 1):
                acc = acc + _lane_perm(acc, sh)
            return acc

        def grp_dot(g, carry):
            r0 = g * 16
            vec = row_total(r0)
            for t in range(1, 16):
                vec = lax.select(lax.eq(lanes, t), row_total(r0 + t), vec)
            scores_v[pl.ds(joff + r0, 16)] = vec
            return carry

        lax.fori_loop(0, CHUNK // 16, grp_dot, 0)
    for j in (N_CHUNKS - 2, N_CHUNKS - 1):
        stores[j][0].wait()
        stores[j][1].wait()
    pltpu.sync_copy(scores_v, sneg_out.at[pl.ds(base, ROWS_PER_W)])


@functools.lru_cache(maxsize=None)
def _make_sc_stage():
    return functools.partial(
        pl.kernel,
        out_type=(
            jax.ShapeDtypeStruct((B, D), jnp.float32),
            jax.ShapeDtypeStruct((B, D), jnp.float32),
            jax.ShapeDtypeStruct((B,), jnp.float32),
        ),
        mesh=plsc.VectorSubcoreMesh(core_axis_name="c", subcore_axis_name="s",
                                    num_cores=NC, num_subcores=NS),
        scratch_types=(
            [pltpu.VMEM((N_CHUNKS, CHUNK), jnp.int32)] * 3
            + [pltpu.VMEM((ROWS_PER_W,), jnp.float32)]
            + [pltpu.VMEM((CHUNK, D), jnp.float32)] * NBUF
            + [pltpu.SemaphoreType.DMA] * (2 * NBUF)
        ),
    )(_sc_body)


BLK = 2048


def _tc_score_body(pos_ref, com_ref, usr_ref, w1_ref, w2_ref, s_ref, sp_ref):
    dn = (((1,), (1,)), ((), ()))
    dncol = (((1,), (0,)), ((), ()))
    fused = (lax.dot_general(pos_ref[...], w1_ref[...], dn,
                             preferred_element_type=jnp.float32)
             + lax.dot_general(com_ref[...], w2_ref[...], dn,
                               preferred_element_type=jnp.float32))
    usr = usr_ref[...]
    out2 = (lax.dot_general(usr * fused, s_ref[0:D, :], dncol,
                            preferred_element_type=jnp.float32)
            + lax.dot_general(usr, s_ref[D:2 * D, :], dncol,
                              preferred_element_type=jnp.float32))
    sp_ref[...] = out2[:, 0]


def _tc_score(pos_emb, comment, user_emb, w1, w2, smat):
    grid = (B // BLK,)
    row_spec = pl.BlockSpec((BLK, D), lambda i: (i, 0))
    full_spec = pl.BlockSpec((D, D), lambda i: (0, 0))
    return pl.pallas_call(
        _tc_score_body,
        grid=grid,
        in_specs=[row_spec, row_spec, row_spec, full_spec, full_spec,
                  pl.BlockSpec((2 * D, 1), lambda i: (0, 0))],
        out_specs=pl.BlockSpec((BLK,), lambda i: (i,)),
        out_shape=jax.ShapeDtypeStruct((B,), jnp.float32),
    )(pos_emb, comment, user_emb, w1, w2, smat)


def kernel(user_ids, positive_product_ids, negative_product_ids,
           positive_comment_embeddings, user_table, product_table, W, b):
    user_emb, pos_emb, score_neg = _make_sc_stage()(
        user_table, product_table,
        user_ids.reshape(B // CHUNK, CHUNK),
        positive_product_ids.reshape(B // CHUNK, CHUNK),
        negative_product_ids.reshape(B // CHUNK, CHUNK))
    w1 = W[:, :D]
    w2 = W[:, D:]
    smat = jnp.concatenate([
        jnp.ones((D, 1), jnp.float32),
        b.reshape(D, 1),
    ], axis=0)
    score_pos = _tc_score(
        pos_emb, positive_comment_embeddings, user_emb, w1, w2, smat)
    return (score_pos, score_neg)


# speedup vs baseline: 1.1691x; 1.1116x over previous
import functools

import jax
import jax.numpy as jnp
from jax import lax
from jax.experimental import pallas as pl
from jax.experimental.pallas import tpu as pltpu
from jax.experimental.pallas import tpu_sc as plsc

NC, NS = 2, 16
NW = NC * NS
B = 16384
D = 128
CHUNK = 128
ROWS_PER_W = B // NW
N_CHUNKS = ROWS_PER_W // CHUNK
NBUF = 7


def _sc_gather_body(user_table, product_table, uid2d, pid2d, nid2d,
                    u_out, p_out, n_out,
                    idx_u, idx_p, idx_n, b0, b1, b2, b3, b4, b5, b6,
                    g0, g1, g2, g3, g4, g5, g6,
                    t0, t1, t2, t3, t4, t5, t6):
    bufs = (b0, b1, b2, b3, b4, b5, b6)
    gsems = (g0, g1, g2, g3, g4, g5, g6)
    ssems = (t0, t1, t2, t3, t4, t5, t6)
    wid = lax.axis_index("s") * NC + lax.axis_index("c")
    base = wid * ROWS_PER_W
    crow = wid * N_CHUNKS
    du = pltpu.async_copy(uid2d.at[pl.ds(crow, N_CHUNKS)], idx_u, t0)
    dp = pltpu.async_copy(pid2d.at[pl.ds(crow, N_CHUNKS)], idx_p, t1)
    dn = pltpu.async_copy(nid2d.at[pl.ds(crow, N_CHUNKS)], idx_n, t2)
    du.wait()
    dp.wait()
    dn.wait()
    tasks = []
    for tbl, idx, out in ((user_table, idx_u, u_out),
                          (product_table, idx_p, p_out),
                          (product_table, idx_n, n_out)):
        for j in range(N_CHUNKS):
            tasks.append((tbl, idx, j, out))
    T = len(tasks)
    LB = NBUF - 1
    gathers = [None] * T
    stores = [None] * T
    for t in range(T + LB):
        if t < T:
            tbl, idx, j, out = tasks[t]
            nb = t % NBUF
            if t >= NBUF:
                stores[t - NBUF].wait()
            gathers[t] = pltpu.async_copy(tbl.at[idx.at[j]], bufs[nb],
                                          gsems[nb])
        u = t - LB
        if 0 <= u < T:
            _, _, j, out = tasks[u]
            gathers[u].wait()
            stores[u] = pltpu.async_copy(
                bufs[u % NBUF], out.at[pl.ds(base + j * CHUNK, CHUNK)],
                ssems[u % NBUF])
    for t in range(max(0, T - NBUF), T):
        stores[t].wait()


@functools.lru_cache(maxsize=None)
def _make_sc_gather():
    return functools.partial(
        pl.kernel,
        out_type=(
            jax.ShapeDtypeStruct((B, D), jnp.float32),
            jax.ShapeDtypeStruct((B, D), jnp.float32),
            jax.ShapeDtypeStruct((B, D), jnp.float32),
        ),
        mesh=plsc.VectorSubcoreMesh(core_axis_name="c", subcore_axis_name="s",
                                    num_cores=NC, num_subcores=NS),
        scratch_types=(
            [pltpu.VMEM((N_CHUNKS, CHUNK), jnp.int32)] * 3
            + [pltpu.VMEM((CHUNK, D), jnp.float32)] * NBUF
            + [pltpu.SemaphoreType.DMA] * (2 * NBUF)
        ),
    )(_sc_gather_body)


BLK = 4096


def _tc_score_body(pos_ref, com_ref, usr_ref, neg_ref, w1_ref, w2_ref, s_ref,
                   sp_ref, sn_ref):
    dn = (((1,), (1,)), ((), ()))
    dncol = (((1,), (0,)), ((), ()))
    fused = (lax.dot_general(pos_ref[...], w1_ref[...], dn,
                             preferred_element_type=jnp.float32)
             + lax.dot_general(com_ref[...], w2_ref[...], dn,
                               preferred_element_type=jnp.float32))
    usr = usr_ref[...]
    out2 = (
        lax.dot_general(usr * fused, s_ref[0:D, :], dncol,
                        preferred_element_type=jnp.float32)
        + lax.dot_general(usr * neg_ref[...], s_ref[D:2 * D, :], dncol,
                          preferred_element_type=jnp.float32)
        + lax.dot_general(usr, s_ref[2 * D:3 * D, :], dncol,
                          preferred_element_type=jnp.float32))
    sp_ref[...] = out2[:, 0]
    sn_ref[...] = out2[:, 1]


def _tc_score(pos_emb, comment, user_emb, neg_emb, w1, w2, smat):
    grid = (B // BLK,)
    row_spec = pl.BlockSpec((BLK, D), lambda i: (i, 0))
    full_spec = pl.BlockSpec((D, D), lambda i: (0, 0))
    return pl.pallas_call(
        _tc_score_body,
        grid=grid,
        in_specs=[row_spec, row_spec, row_spec, row_spec,
                  full_spec, full_spec,
                  pl.BlockSpec((3 * D, 2), lambda i: (0, 0))],
        out_specs=[pl.BlockSpec((BLK,), lambda i: (i,)),
                   pl.BlockSpec((BLK,), lambda i: (i,))],
        out_shape=[jax.ShapeDtypeStruct((B,), jnp.float32),
                   jax.ShapeDtypeStruct((B,), jnp.float32)],
    )(pos_emb, comment, user_emb, neg_emb, w1, w2, smat)


def kernel(user_ids, positive_product_ids, negative_product_ids,
           positive_comment_embeddings, user_table, product_table, W, b):
    user_emb, pos_emb, neg_emb = _make_sc_gather()(
        user_table, product_table,
        user_ids.reshape(B // CHUNK, CHUNK),
        positive_product_ids.reshape(B // CHUNK, CHUNK),
        negative_product_ids.reshape(B // CHUNK, CHUNK))
    w1 = W[:, :D]
    w2 = W[:, D:]
    smat = jnp.concatenate([
        jnp.tile(jnp.array([[1.0, 0.0]], jnp.float32), (D, 1)),
        jnp.tile(jnp.array([[0.0, 1.0]], jnp.float32), (D, 1)),
        jnp.stack([b, jnp.zeros_like(b)], axis=1),
    ], axis=0)
    score_pos, score_neg = _tc_score(
        pos_emb, positive_comment_embeddings, user_emb, neg_emb, w1, w2, smat)
    return (score_pos, score_neg)
